# baseline (speedup 1.0000x reference)
import functools

import jax
import jax.numpy as jnp
from jax import lax
from jax.experimental import pallas as pl
from jax.experimental.pallas import tpu as pltpu
from jax.experimental.pallas import tpu_sc as plsc

_NUM_CORES = 2
_NUM_SUBCORES = 16
_NW = _NUM_CORES * _NUM_SUBCORES
_LANES = 16
_IC = 32
_ACCS = 4


@functools.lru_cache(maxsize=None)
def _make_sc_hinge(B: int, C: int):
    R = B // _NW
    n_chunks = R // _IC
    n_groups = _IC // _LANES
    n_full = C // _LANES
    rem = C - n_full * _LANES

    mesh = plsc.VectorSubcoreMesh(core_axis_name="c", subcore_axis_name="s")

    @functools.partial(
        pl.kernel,
        mesh=mesh,
        out_type=jax.ShapeDtypeStruct((B,), jnp.float32),
        compiler_params=pltpu.CompilerParams(needs_layout_passes=False),
        scratch_types=[
            pltpu.VMEM((R,), jnp.int32),
            pltpu.VMEM((R,), jnp.float32),
            pltpu.VMEM((_IC,), jnp.float32),
            pltpu.VMEM((_IC, C), jnp.float32),
            pltpu.VMEM((_IC, C), jnp.float32),
            pltpu.SemaphoreType.DMA,
            pltpu.SemaphoreType.DMA,
        ],
    )
    def sc_hinge(x_hbm, y_hbm, out_hbm, y_v, out_v, a_v, xb0, xb1,
                 sem0, sem1):
        wid = lax.axis_index("s") * _NUM_CORES + lax.axis_index("c")
        base = wid * R
        pltpu.sync_copy(y_hbm.at[pl.ds(base, R)], y_v)

        bufs = (xb0, xb1)
        sems = (sem0, sem1)

        def start_slab(ci, b):
            pltpu.async_copy(
                x_hbm.at[pl.ds(base + ci * _IC, _IC), :],
                bufs[b],
                sems[b],
            )

        def wait_slab(b):
            pltpu.make_async_copy(
                x_hbm.at[pl.ds(0, _IC), :], bufs[b], sems[b]
            ).wait()

        lanes = lax.iota(jnp.int32, _LANES)
        zero = jnp.zeros((_LANES,), jnp.float32)
        last_lane = lanes == (_LANES - 1)
        tail_keep = lanes >= (_LANES - rem)

        def compute(ci, buf):
            for g in range(n_groups):
                col = g * _LANES
                yv = y_v[pl.ds(ci * _IC + col, _LANES)]
                t_vec = plsc.load_gather(buf, [lanes + col, yv])
                a_v[pl.ds(col, _LANES)] = t_vec - 1.0

            def row_body(r, _):
                rv = jnp.full((_LANES,), r, jnp.int32)
                b_splat = plsc.load_gather(a_v, [rv])
                accs = [zero] * _ACCS
                for c in range(n_full):
                    x = buf[r, pl.ds(c * _LANES, _LANES)]
                    accs[c % _ACCS] = accs[c % _ACCS] + jnp.maximum(
                        x, b_splat
                    )
                acc = (accs[0] + accs[1]) + (accs[2] + accs[3])
                if rem:
                    x = buf[r, pl.ds(C - _LANES, _LANES)]
                    h = jnp.maximum(x, b_splat)
                    acc = acc + jnp.where(tail_keep, h, 0.0)
                total = plsc.cumsum(acc)
                loss = (total - 1.0) * (1.0 / C) - b_splat
                plsc.store_scatter(
                    out_v, [rv + ci * _IC], loss, mask=last_lane
                )
                return _

            plsc.parallel_loop(0, _IC, unroll=4, carry=jnp.int32(0))(
                row_body
            )

        start_slab(0, 0)
        start_slab(1, 1)

        @pl.loop(0, n_chunks - 2, step=2)
        def _slab_ring(cil):
            for b in range(2):
                wait_slab(b)
                compute(cil + b, bufs[b])
                start_slab(cil + b + 2, b)

        for b in range(2):
            wait_slab(b)
            compute(n_chunks - 2 + b, bufs[b])

        pltpu.sync_copy(out_v, out_hbm.at[pl.ds(base, R)])

    return sc_hinge


@jax.jit
def kernel(output, y):
    B, C = output.shape
    y32 = y.astype(jnp.int32)
    return _make_sc_hinge(B, C)(output, y32)


# speedup vs baseline: 1.1380x; 1.1380x over previous
import functools

import jax
import jax.numpy as jnp
from jax import lax
from jax.experimental import pallas as pl
from jax.experimental.pallas import tpu as pltpu
from jax.experimental.pallas import tpu_sc as plsc

_NUM_CORES = 2
_NUM_SUBCORES = 16
_NW = _NUM_CORES * _NUM_SUBCORES
_LANES = 16
_IC = 32
_ACCS = 4


@functools.lru_cache(maxsize=None)
def _make_sc_hinge(B: int, C: int):
    R = B // _NW
    n_chunks = R // _IC
    n_groups = _IC // _LANES
    n_full = C // _LANES
    rem = C - n_full * _LANES

    mesh = plsc.VectorSubcoreMesh(core_axis_name="c", subcore_axis_name="s")

    @functools.partial(
        pl.kernel,
        mesh=mesh,
        out_type=jax.ShapeDtypeStruct((B,), jnp.float32),
        compiler_params=pltpu.CompilerParams(needs_layout_passes=False),
        scratch_types=[
            pltpu.VMEM((R,), jnp.int32),
            pltpu.VMEM((R,), jnp.float32),
            pltpu.VMEM((_IC,), jnp.float32),
            pltpu.VMEM((_IC, C), jnp.float32),
            pltpu.VMEM((_IC, C), jnp.float32),
            pltpu.SemaphoreType.DMA,
            pltpu.SemaphoreType.DMA,
        ],
    )
    def sc_hinge(x_hbm, y_hbm, out_hbm, y_v, out_v, a_v, xb0, xb1,
                 sem0, sem1):
        wid = lax.axis_index("s") * _NUM_CORES + lax.axis_index("c")
        base = wid * R
        pltpu.sync_copy(y_hbm.at[pl.ds(base, R)], y_v)

        bufs = (xb0, xb1)
        sems = (sem0, sem1)

        def start_slab(ci, b):
            pltpu.async_copy(
                x_hbm.at[pl.ds(base + ci * _IC, _IC), :],
                bufs[b],
                sems[b],
            )

        def wait_slab(b):
            pltpu.make_async_copy(
                x_hbm.at[pl.ds(0, _IC), :], bufs[b], sems[b]
            ).wait()

        lanes = lax.iota(jnp.int32, _LANES)
        zero = jnp.zeros((_LANES,), jnp.float32)
        last_lane = lanes == (_LANES - 1)
        tail_keep = lanes >= (_LANES - rem)

        def compute(ci, buf):
            for g in range(n_groups):
                col = g * _LANES
                yv = y_v[pl.ds(ci * _IC + col, _LANES)]
                t_vec = plsc.load_gather(buf, [lanes + col, yv])
                a_v[pl.ds(col, _LANES)] = 1.0 - t_vec

            def row_body(r, _):
                rv = jnp.full((_LANES,), r, jnp.int32)
                a_splat = plsc.load_gather(a_v, [rv])
                accs = [zero] * _ACCS
                for c in range(n_full):
                    x = buf[r, pl.ds(c * _LANES, _LANES)]
                    accs[c % _ACCS] = accs[c % _ACCS] + jnp.maximum(
                        x + a_splat, 0.0
                    )
                acc = (accs[0] + accs[1]) + (accs[2] + accs[3])
                if rem:
                    x = buf[r, pl.ds(C - _LANES, _LANES)]
                    h = jnp.maximum(x + a_splat, 0.0)
                    acc = acc + jnp.where(tail_keep, h, 0.0)
                total = plsc.cumsum(acc)
                loss = (total - 1.0) * (1.0 / C)
                plsc.store_scatter(
                    out_v, [rv + ci * _IC], loss, mask=last_lane
                )
                return _

            plsc.parallel_loop(0, _IC, unroll=4, carry=jnp.int32(0))(
                row_body
            )

        start_slab(0, 0)
        start_slab(1, 1)

        @pl.loop(0, n_chunks - 2, step=2)
        def _slab_ring(cil):
            for b in range(2):
                wait_slab(b)
                compute(cil + b, bufs[b])
                start_slab(cil + b + 2, b)

        for b in range(2):
            wait_slab(b)
            compute(n_chunks - 2 + b, bufs[b])

        pltpu.sync_copy(out_v, out_hbm.at[pl.ds(base, R)])

    return sc_hinge


@jax.jit
def kernel(output, y):
    B, C = output.shape
    y32 = y.astype(jnp.int32)
    return _make_sc_hinge(B, C)(output, y32)
